# baseline (speedup 1.0000x reference)
import functools

import jax
import jax.numpy as jnp
from jax import lax
from jax.experimental import pallas as pl
from jax.experimental.pallas import tpu as pltpu
from jax.experimental.pallas import tpu_sc as plsc

N = 10000
E = 320000
D = 128
G = 16
L = 2

NC = 2
NS = 16
NW = NC * NS
EW = E // NW
CH = 80
NCHUNK = EW // CH
RS = 624
RT = N - NS * RS

BN = 2000
BE = 2000

f32 = jnp.float32
i32 = jnp.int32


def _ln(x, g, b):
    m = jnp.mean(x, axis=-1, keepdims=True)
    v = jnp.mean((x - m) * (x - m), axis=-1, keepdims=True)
    return (x - m) * lax.rsqrt(v + 1e-5) * g + b


def _embed_body(a_ref, emb_ref, out_ref):
    a = a_ref[0]
    oh = (a == lax.broadcasted_iota(i32, (BN, D), 1)).astype(f32)
    out_ref[...] = jnp.dot(oh, emb_ref[...], preferred_element_type=f32)


def _embed(atoms3, embp):
    return pl.pallas_call(
        _embed_body,
        grid=(N // BN,),
        in_specs=[
            pl.BlockSpec((1, BN, 1), lambda i: (i, 0, 0)),
            pl.BlockSpec((D, D), lambda i: (0, 0)),
        ],
        out_specs=pl.BlockSpec((BN, D), lambda i: (i, 0)),
        out_shape=jax.ShapeDtypeStruct((N, D), f32),
    )(atoms3, embp)


def _gather_body(plan, nt, *refs):
    ns = len(plan)
    tables = refs[:nt]
    src_hbm, dst_hbm = refs[nt], refs[nt + 1]
    outs = refs[nt + 2:nt + 2 + ns]
    (idxs_big, idxd_big, bufs0, bufs1,
     gsem0, gsem1, wsem0, wsem1) = refs[nt + 2 + ns:]
    wid = lax.axis_index("s") * NC + lax.axis_index("c")
    wbase = wid * EW

    def fill_idx(k, carry):
        base = pl.multiple_of(wbase + k * CH, CH)
        pltpu.async_copy(src_hbm.at[pl.ds(base, CH)], idxs_big.at[k], gsem0)
        pltpu.async_copy(dst_hbm.at[pl.ds(base, CH)], idxd_big.at[k], gsem0)
        return carry

    lax.fori_loop(0, NCHUNK, fill_idx, 0)

    def drain_idx(k, carry):
        base = pl.multiple_of(wbase + k * CH, CH)
        pltpu.make_async_copy(src_hbm.at[pl.ds(base, CH)],
                              idxs_big.at[k], gsem0).wait()
        pltpu.make_async_copy(dst_hbm.at[pl.ds(base, CH)],
                              idxd_big.at[k], gsem0).wait()
        return carry

    lax.fori_loop(0, NCHUNK, drain_idx, 0)

    sets = ((bufs0, gsem0, wsem0), (bufs1, gsem1, wsem1))

    def srcs(k):
        res = []
        for ti, side in plan:
            ib = idxs_big if side else idxd_big
            res.append(tables[ti].at[ib.at[k]])
        return res

    def issue_gather(k, st):
        bufs, gs, _ = st
        for sref, b in zip(srcs(k), bufs):
            pltpu.async_copy(sref, b, gs)

    def drain_gather(k, st):
        bufs, gs, _ = st
        for sref, b in zip(srcs(k), bufs):
            pltpu.make_async_copy(sref, b, gs).wait()

    def issue_write(k, st):
        bufs, _, ws = st
        base = pl.multiple_of(wbase + k * CH, CH)
        for o, b in zip(outs, bufs):
            pltpu.async_copy(b, o.at[pl.ds(base, CH)], ws)

    def drain_write(k, st):
        bufs, _, ws = st
        base = pl.multiple_of(wbase + k * CH, CH)
        for o, b in zip(outs, bufs):
            pltpu.make_async_copy(b, o.at[pl.ds(base, CH)], ws).wait()

    issue_gather(0, sets[0])
    issue_gather(1, sets[1])

    def body(g, carry):
        k0 = 2 * g
        drain_gather(k0, sets[0])
        issue_write(k0, sets[0])
        drain_gather(k0 + 1, sets[1])
        issue_write(k0 + 1, sets[1])
        drain_write(k0, sets[0])
        issue_gather(k0 + 2, sets[0])

        @pl.when(k0 + 3 < NCHUNK)
        def _():
            drain_write(k0 + 1, sets[1])
            issue_gather(k0 + 3, sets[1])

        return carry

    lax.fori_loop(0, (NCHUNK - 1) // 2, body, 0)
    last = NCHUNK - 1
    drain_write(last - 1, sets[1])
    drain_gather(last, sets[0])
    issue_write(last, sets[0])
    drain_write(last, sets[0])


def _run_gather(tables, src, dst, plan):
    mesh = plsc.VectorSubcoreMesh(core_axis_name="c", subcore_axis_name="s")
    out_type = []
    bufset = []
    for ti, _ in plan:
        t = tables[ti]
        if t.ndim == 2:
            out_type.append(jax.ShapeDtypeStruct((E, t.shape[1]), t.dtype))
            bufset.append(pltpu.VMEM((CH, t.shape[1]), t.dtype))
        else:
            out_type.append(jax.ShapeDtypeStruct((E,), t.dtype))
            bufset.append(pltpu.VMEM((CH,), t.dtype))
    fn = pl.kernel(
        functools.partial(_gather_body, plan, len(tables)),
        out_type=out_type,
        mesh=mesh,
        scratch_types=[
            pltpu.VMEM((NCHUNK, CH), i32),
            pltpu.VMEM((NCHUNK, CH), i32),
            list(bufset),
            list(bufset),
            pltpu.SemaphoreType.DMA,
            pltpu.SemaphoreType.DMA,
            pltpu.SemaphoreType.DMA,
            pltpu.SemaphoreType.DMA,
        ],
    )
    return fn(*tables, src, dst)


_PLAN_SIDES = [(0, 0), (0, 1), (1, 0), (2, 0), (3, 0), (1, 1), (2, 1), (3, 1)]


def _gather0(atoms_i, px, py, pz, src, dst):
    return _run_gather([atoms_i, px, py, pz], src, dst, _PLAN_SIDES)


def _gather1(h, px, py, pz, src, dst):
    return _run_gather([h, px, py, pz], src, dst, _PLAN_SIDES)


def _scatter_body(has_pos, *refs):
    if has_pos:
        (msg_hbm, posw_hbm, dst_hbm, zn_hbm, mp_out, pp_out,
         acc, bmsg0, bmsg1, bidx0, bidx1, lsem0, lsem1) = refs
    else:
        (msg_hbm, dst_hbm, zn_hbm, mp_out,
         acc, bmsg0, bmsg1, bidx0, bidx1, lsem0, lsem1) = refs
    c = lax.axis_index("c")
    s = lax.axis_index("s")
    r0 = s * RS
    wid = s * NC + c
    wbase = wid * EW

    def one_pass(data_hbm, out_hbm):
        pltpu.sync_copy(zn_hbm.at[pl.ds(r0, RS)], acc.at[pl.ds(r0, RS)])

        @pl.when(s == 0)
        def _():
            pltpu.sync_copy(zn_hbm.at[pl.ds(NS * RS, RT)],
                            acc.at[pl.ds(NS * RS, RT)])

        plsc.subcore_barrier()

        sets = ((bmsg0, bidx0, lsem0), (bmsg1, bidx1, lsem1))

        def issue_load(k, st):
            bm, bi, ls = st
            base = pl.multiple_of(wbase + k * CH, CH)
            pltpu.async_copy(data_hbm.at[pl.ds(base, CH)], bm, ls)
            pltpu.async_copy(dst_hbm.at[pl.ds(base, CH)], bi, ls)

        def do_scatter(k, st):
            bm, bi, ls = st
            base = pl.multiple_of(wbase + k * CH, CH)
            pltpu.make_async_copy(data_hbm.at[pl.ds(base, CH)], bm, ls).wait()
            pltpu.make_async_copy(dst_hbm.at[pl.ds(base, CH)], bi, ls).wait()
            pltpu.sync_copy(bm, acc.at[bi], add=True)

        issue_load(0, sets[0])
        issue_load(1, sets[1])

        def body(g, carry):
            k0 = 2 * g
            do_scatter(k0, sets[0])
            issue_load(k0 + 2, sets[0])
            do_scatter(k0 + 1, sets[1])

            @pl.when(k0 + 3 < NCHUNK)
            def _():
                issue_load(k0 + 3, sets[1])

            return carry

        lax.fori_loop(0, (NCHUNK - 1) // 2, body, 0)
        do_scatter(NCHUNK - 1, sets[0])

        plsc.subcore_barrier()
        pltpu.sync_copy(acc.at[pl.ds(r0, RS)], out_hbm.at[c, pl.ds(r0, RS)])

        @pl.when(s == 0)
        def _():
            pltpu.sync_copy(acc.at[pl.ds(NS * RS, RT)],
                            out_hbm.at[c, pl.ds(NS * RS, RT)])

    one_pass(msg_hbm, mp_out)
    if has_pos:
        one_pass(posw_hbm, pp_out)


def _scatter(msg, dst, zn, posw=None):
    mesh = plsc.VectorSubcoreMesh(core_axis_name="c", subcore_axis_name="s")
    has_pos = posw is not None
    out_type = [jax.ShapeDtypeStruct((NC, N, D), f32)]
    args = [msg]
    if has_pos:
        out_type.append(jax.ShapeDtypeStruct((NC, N, D), f32))
        args.append(posw)
    args += [dst, zn]
    scratch = [
        pltpu.VMEM_SHARED((N, D), f32),
        pltpu.VMEM((CH, D), f32),
        pltpu.VMEM((CH, D), f32),
        pltpu.VMEM((CH,), i32),
        pltpu.VMEM((CH,), i32),
        pltpu.SemaphoreType.DMA,
        pltpu.SemaphoreType.DMA,
    ]
    fn = pl.kernel(
        functools.partial(_scatter_body, has_pos),
        out_type=out_type,
        mesh=mesh,
        scratch_types=scratch,
    )
    return fn(*args)


def _edge_tail(t, g1_ref, be1_ref, w2_ref, b2_ref, g2_ref, be2_ref):
    x1 = jax.nn.relu(_ln(t, g1_ref[...], be1_ref[...]))
    return jax.nn.relu(_ln(jnp.dot(x1, w2_ref[...], preferred_element_type=f32)
                           + b2_ref[...], g2_ref[...], be2_ref[...]))


def _dist_cols(prefs):
    pdx, pdy, pdz, psx, psy, psz = [r[0] for r in prefs]
    dx = pdx - psx
    dy = pdy - psy
    dz = pdz - psz
    dist = jnp.sqrt(dx * dx + dy * dy + dz * dz)
    return dx, dy, dz, dist


def _edge0_body(ad_ref, as_ref, pdx_ref, pdy_ref, pdz_ref,
                psx_ref, psy_ref, psz_ref, embp_ref,
                w1d_ref, w1s_ref, w1x_ref, b1_ref, g1_ref, be1_ref,
                w2_ref, b2_ref, g2_ref, be2_ref,
                pw1_ref, pb1_ref, pg_ref, pbe_ref, pw2_ref, pb2_ref,
                msg_ref, posw_ref):
    dx, dy, dz, dist = _dist_cols(
        (pdx_ref, pdy_ref, pdz_ref, psx_ref, psy_ref, psz_ref))
    lane = lax.broadcasted_iota(i32, (BE, D), 1)
    ohd = (ad_ref[0] == lane).astype(f32)
    ohs = (as_ref[0] == lane).astype(f32)
    ew1d = jnp.dot(embp_ref[...], w1d_ref[...], preferred_element_type=f32)
    ew1s = jnp.dot(embp_ref[...], w1s_ref[...], preferred_element_type=f32)
    t = (jnp.dot(ohd, ew1d, preferred_element_type=f32)
         + jnp.dot(ohs, ew1s, preferred_element_type=f32)
         + dist * w1x_ref[...] + b1_ref[...])
    msg = _edge_tail(t, g1_ref, be1_ref, w2_ref, b2_ref, g2_ref, be2_ref)
    msg_ref[...] = msg
    y = jax.nn.relu(_ln(jnp.dot(msg, pw1_ref[...], preferred_element_type=f32)
                        + pb1_ref[...], pg_ref[...], pbe_ref[...]))
    sval = jnp.sum(y * pw2_ref[...], axis=1, keepdims=True) + pb2_ref[...]
    posw_ref[...] = (jnp.where(lane == 0, dx * sval, 0.0)
                     + jnp.where(lane == 1, dy * sval, 0.0)
                     + jnp.where(lane == 2, dz * sval, 0.0)
                     + (lane == 3).astype(f32))


def _edge1_body(hd_ref, hs_ref, pdx_ref, pdy_ref, pdz_ref,
                psx_ref, psy_ref, psz_ref,
                w1d_ref, w1s_ref, w1x_ref, b1_ref, g1_ref, be1_ref,
                w2_ref, b2_ref, g2_ref, be2_ref, msg_ref):
    _, _, _, dist = _dist_cols(
        (pdx_ref, pdy_ref, pdz_ref, psx_ref, psy_ref, psz_ref))
    t = (jnp.dot(hd_ref[...], w1d_ref[...], preferred_element_type=f32)
         + jnp.dot(hs_ref[...], w1s_ref[...], preferred_element_type=f32)
         + dist * w1x_ref[...] + b1_ref[...])
    msg_ref[...] = _edge_tail(t, g1_ref, be1_ref, w2_ref, b2_ref,
                              g2_ref, be2_ref)


_espec = pl.BlockSpec((BE, D), lambda i: (i, 0))
_cspec = pl.BlockSpec((1, BE, 1), lambda i: (i, 0, 0))
_wspec = pl.BlockSpec((D, D), lambda i: (0, 0))
_rspec = pl.BlockSpec((1, D), lambda i: (0, 0))


def _edge0(ad3, as3, pcols3, embp, w1d, w1s, w1x, b1, g1, be1,
           w2, b2, g2, be2, pos_head):
    pw1, pb1, pg, pbe, pw2, pb2 = pos_head
    in_specs = ([_cspec] * 8 + [_wspec, _wspec, _wspec, _rspec, _rspec,
                _rspec, _rspec, _wspec, _rspec, _rspec, _rspec,
                _wspec, _rspec, _rspec, _rspec, _rspec,
                pl.BlockSpec((1, 1), lambda i: (0, 0))])
    return pl.pallas_call(
        _edge0_body,
        grid=(E // BE,),
        in_specs=in_specs,
        out_specs=[_espec, _espec],
        out_shape=[jax.ShapeDtypeStruct((E, D), f32),
                   jax.ShapeDtypeStruct((E, D), f32)],
    )(ad3, as3, *pcols3, embp, w1d, w1s, w1x, b1, g1, be1,
      w2, b2, g2, be2, pw1, pb1, pg, pbe, pw2, pb2)


def _edge1(hd, hs, pcols3, w1d, w1s, w1x, b1, g1, be1, w2, b2, g2, be2):
    in_specs = ([_espec, _espec] + [_cspec] * 6
                + [_wspec, _wspec, _rspec, _rspec, _rspec, _rspec,
                   _wspec, _rspec, _rspec, _rspec])
    return pl.pallas_call(
        _edge1_body,
        grid=(E // BE,),
        in_specs=in_specs,
        out_specs=_espec,
        out_shape=jax.ShapeDtypeStruct((E, D), f32),
    )(hd, hs, *pcols3, w1d, w1s, w1x, b1, g1, be1, w2, b2, g2, be2)


def _node_body(has_pos, h_ref, mp_ref,
               w1h_ref, w1m_ref, b1_ref, g1_ref, be1_ref,
               w2_ref, b2_ref, g2_ref, be2_ref, *rest):
    h = h_ref[...]
    ma = mp_ref[0] + mp_ref[1]
    t = (jnp.dot(h, w1h_ref[...], preferred_element_type=f32)
         + jnp.dot(ma, w1m_ref[...], preferred_element_type=f32) + b1_ref[...])
    u = jax.nn.relu(_ln(t, g1_ref[...], be1_ref[...]))
    upd = jax.nn.relu(_ln(jnp.dot(u, w2_ref[...], preferred_element_type=f32)
                          + b2_ref[...], g2_ref[...], be2_ref[...]))
    if has_pos:
        pp_ref, pos_ref, ho_ref, po_ref = rest
        ho_ref[...] = h + upd
        psum = pp_ref[0] + pp_ref[1]
        lane = lax.broadcasted_iota(i32, (BN, D), 1)
        cnt = jnp.sum(jnp.where(lane == 3, psum, 0.0), axis=1, keepdims=True)
        po_ref[...] = pos_ref[...] + (
            jnp.where(lane < 3, psum, 0.0) / jnp.maximum(cnt, 1.0))
    else:
        (ho_ref,) = rest
        ho_ref[...] = h + upd


def _node(h, mparts, w1h, w1m, b1, g1, be1, w2, b2, g2, be2,
          pos_part=None):
    wspec = pl.BlockSpec((D, D), lambda i: (0, 0))
    rspec = pl.BlockSpec((1, D), lambda i: (0, 0))
    in_specs = [
        pl.BlockSpec((BN, D), lambda i: (i, 0)),
        pl.BlockSpec((NC, BN, D), lambda i: (0, i, 0)),
        wspec, wspec, rspec, rspec, rspec, wspec, rspec, rspec, rspec,
    ]
    args = [h, mparts, w1h, w1m, b1, g1, be1, w2, b2, g2, be2]
    if pos_part is not None:
        pparts, pp = pos_part
        in_specs += [pl.BlockSpec((NC, BN, D), lambda i: (0, i, 0)),
                     pl.BlockSpec((BN, D), lambda i: (i, 0))]
        args += [pparts, pp]
        out_specs = [pl.BlockSpec((BN, D), lambda i: (i, 0)),
                     pl.BlockSpec((BN, D), lambda i: (i, 0))]
        out_shape = [jax.ShapeDtypeStruct((N, D), f32),
                     jax.ShapeDtypeStruct((N, D), f32)]
    else:
        out_specs = pl.BlockSpec((BN, D), lambda i: (i, 0))
        out_shape = jax.ShapeDtypeStruct((N, D), f32)
    return pl.pallas_call(
        functools.partial(_node_body, pos_part is not None),
        grid=(N // BN,),
        in_specs=in_specs,
        out_specs=out_specs,
        out_shape=out_shape,
    )(*args)


def _pool_body(h_ref, b_ref, w1_ref, b1_ref, w2t_ref, b2_ref, out_ref):
    b = b_ref[...]
    oh = (lax.broadcasted_iota(i32, (G, N), 0) == b).astype(f32)
    pooled = jnp.dot(oh, h_ref[...], preferred_element_type=f32)
    r = jax.nn.relu(jnp.dot(pooled, w1_ref[...], preferred_element_type=f32)
                    + b1_ref[...])
    out_ref[...] = jnp.sum(r * w2t_ref[...], axis=1, keepdims=True) + b2_ref[...]


def _pool(h, batch2, w1, b1, w2t, b2):
    return pl.pallas_call(
        _pool_body,
        grid=(1,),
        in_specs=[
            pl.BlockSpec((N, D), lambda i: (0, 0)),
            pl.BlockSpec((1, N), lambda i: (0, 0)),
            pl.BlockSpec((D, D), lambda i: (0, 0)),
            pl.BlockSpec((1, D), lambda i: (0, 0)),
            pl.BlockSpec((1, D), lambda i: (0, 0)),
            pl.BlockSpec((1, 1), lambda i: (0, 0)),
        ],
        out_specs=pl.BlockSpec((G, 1), lambda i: (0, 0)),
        out_shape=jax.ShapeDtypeStruct((G, 1), f32),
    )(h, batch2, w1, b1, w2t, b2)


def kernel(atoms, pos, edge_index, batch, emb,
           msg_W1, msg_b1, msg_g1, msg_be1, msg_W2, msg_b2, msg_g2, msg_be2,
           pos_W1, pos_b1, pos_g, pos_be, pos_W2, pos_b2,
           upd_W1, upd_b1, upd_g1, upd_be1, upd_W2, upd_b2, upd_g2, upd_be2,
           pred_W1, pred_b1, pred_W2, pred_b2):
    src = edge_index[0].astype(i32)
    dst = edge_index[1].astype(i32)
    posp = jnp.pad(pos.astype(f32), ((0, 0), (0, D - 3)))
    embp = jnp.pad(emb, ((0, D - emb.shape[0]), (0, 0)))
    atoms_i = atoms.astype(i32)
    atoms3 = atoms_i.reshape(N // BN, BN, 1)
    batch2 = batch.astype(i32).reshape(1, N)
    zn = jnp.zeros((N, D), f32)

    def r1(a):
        return a.reshape(1, D)

    def rc(a):
        return a.reshape(E // BE, BE, 1)

    def msg_w(l):
        return (msg_W1[l, :D], msg_W1[l, D:2 * D], msg_W1[l, 2 * D:2 * D + 1],
                r1(msg_b1[l]), r1(msg_g1[l]), r1(msg_be1[l]), msg_W2[l],
                r1(msg_b2[l]), r1(msg_g2[l]), r1(msg_be2[l]))

    def upd_w(l):
        return (upd_W1[l, :D], upd_W1[l, D:], r1(upd_b1[l]), r1(upd_g1[l]),
                r1(upd_be1[l]), upd_W2[l], r1(upd_b2[l]), r1(upd_g2[l]),
                r1(upd_be2[l]))

    h = _embed(atoms3, embp)

    px0, py0, pz0 = pos[:, 0], pos[:, 1], pos[:, 2]
    ad, as_, *pcols = _gather0(atoms_i, px0, py0, pz0, src, dst)
    pos_head = (pos_W1[0], r1(pos_b1[0]), r1(pos_g[0]), r1(pos_be[0]),
                pos_W2[0].reshape(1, D), pos_b2[0].reshape(1, 1))
    msg, posw = _edge0(rc(ad), rc(as_), [rc(p) for p in pcols], embp,
                       *msg_w(0), pos_head)
    mparts, pparts = _scatter(msg, dst, zn, posw=posw)
    h, pp = _node(h, mparts, *upd_w(0), pos_part=(pparts, posp))

    px1, py1, pz1 = pp[:, 0], pp[:, 1], pp[:, 2]
    hd, hs, *pcols = _gather1(h, px1, py1, pz1, src, dst)
    msg = _edge1(hd, hs, [rc(p) for p in pcols], *msg_w(1))
    (mparts,) = _scatter(msg, dst, zn)
    h = _node(h, mparts, *upd_w(1))

    return _pool(h, batch2, pred_W1, pred_b1.reshape(1, D),
                 pred_W2.reshape(1, D), pred_b2.reshape(1, 1))


# speedup vs baseline: 2.3004x; 2.3004x over previous
import functools

import jax
import jax.numpy as jnp
from jax import lax
from jax.experimental import pallas as pl
from jax.experimental.pallas import tpu as pltpu
from jax.experimental.pallas import tpu_sc as plsc

N = 10000
E = 320000
D = 128
G = 16
L = 2

NC = 2
NS = 16
NW = NC * NS
EW = E // NW
CH = 80
NCHUNK = EW // CH
RS = 624
RT = N - NS * RS

BN = 2000
BE = 2000

f32 = jnp.float32
i32 = jnp.int32


def _ln(x, g, b):
    m = jnp.mean(x, axis=-1, keepdims=True)
    v = jnp.mean((x - m) * (x - m), axis=-1, keepdims=True)
    return (x - m) * lax.rsqrt(v + 1e-5) * g + b


def _embed_body(a_ref, emb_ref, out_ref):
    a = a_ref[0]
    oh = (a == lax.broadcasted_iota(i32, (BN, D), 1)).astype(f32)
    out_ref[...] = jnp.dot(oh, emb_ref[...], preferred_element_type=f32)


def _embed(atoms3, embp):
    return pl.pallas_call(
        _embed_body,
        grid=(N // BN,),
        in_specs=[
            pl.BlockSpec((1, BN, 1), lambda i: (i, 0, 0)),
            pl.BlockSpec((D, D), lambda i: (0, 0)),
        ],
        out_specs=pl.BlockSpec((BN, D), lambda i: (i, 0)),
        out_shape=jax.ShapeDtypeStruct((N, D), f32),
    )(atoms3, embp)


def _gather_body(h_hbm, pp_hbm, src_hbm, dst_hbm,
                 hd_out, hs_out, pd_out, ps_out,
                 idxs_big, idxd_big, bufs0, bufs1,
                 gsem0, gsem1, wsem0, wsem1):
    wid = lax.axis_index("s") * NC + lax.axis_index("c")
    wbase = wid * EW

    def fill_idx(k, carry):
        base = pl.multiple_of(wbase + k * CH, CH)
        pltpu.async_copy(src_hbm.at[pl.ds(base, CH)], idxs_big.at[k], gsem0)
        pltpu.async_copy(dst_hbm.at[pl.ds(base, CH)], idxd_big.at[k], gsem0)
        return carry

    lax.fori_loop(0, NCHUNK, fill_idx, 0)

    def drain_idx(k, carry):
        base = pl.multiple_of(wbase + k * CH, CH)
        pltpu.make_async_copy(src_hbm.at[pl.ds(base, CH)],
                              idxs_big.at[k], gsem0).wait()
        pltpu.make_async_copy(dst_hbm.at[pl.ds(base, CH)],
                              idxd_big.at[k], gsem0).wait()
        return carry

    lax.fori_loop(0, NCHUNK, drain_idx, 0)

    sets = ((bufs0, gsem0, wsem0), (bufs1, gsem1, wsem1))
    outs = (hd_out, hs_out, pd_out, ps_out)

    def srcs(k):
        return (h_hbm.at[idxd_big.at[k]],
                h_hbm.at[idxs_big.at[k]],
                pp_hbm.at[idxd_big.at[k]],
                pp_hbm.at[idxs_big.at[k]])

    def issue_gather(k, st):
        bufs, gs, _ = st
        for sref, b in zip(srcs(k), bufs):
            pltpu.async_copy(sref, b, gs)

    def drain_gather(k, st):
        bufs, gs, _ = st
        for sref, b in zip(srcs(k), bufs):
            pltpu.make_async_copy(sref, b, gs).wait()

    def issue_write(k, st):
        bufs, _, ws = st
        base = pl.multiple_of(wbase + k * CH, CH)
        for o, b in zip(outs, bufs):
            pltpu.async_copy(b, o.at[pl.ds(base, CH)], ws)

    def drain_write(k, st):
        bufs, _, ws = st
        base = pl.multiple_of(wbase + k * CH, CH)
        for o, b in zip(outs, bufs):
            pltpu.make_async_copy(b, o.at[pl.ds(base, CH)], ws).wait()

    issue_gather(0, sets[0])
    issue_gather(1, sets[1])

    def body(g, carry):
        k0 = 2 * g
        drain_gather(k0, sets[0])
        issue_write(k0, sets[0])
        drain_gather(k0 + 1, sets[1])
        issue_write(k0 + 1, sets[1])
        drain_write(k0, sets[0])
        issue_gather(k0 + 2, sets[0])

        @pl.when(k0 + 3 < NCHUNK)
        def _():
            drain_write(k0 + 1, sets[1])
            issue_gather(k0 + 3, sets[1])

        return carry

    lax.fori_loop(0, (NCHUNK - 1) // 2, body, 0)
    last = NCHUNK - 1
    drain_write(last - 1, sets[1])
    drain_gather(last, sets[0])
    issue_write(last, sets[0])
    drain_write(last, sets[0])


def _gather(h, pp, src, dst):
    mesh = plsc.VectorSubcoreMesh(core_axis_name="c", subcore_axis_name="s")
    fn = pl.kernel(
        _gather_body,
        out_type=[
            jax.ShapeDtypeStruct((E, D), f32),
            jax.ShapeDtypeStruct((E, D), f32),
            jax.ShapeDtypeStruct((E, D), f32),
            jax.ShapeDtypeStruct((E, D), f32),
        ],
        mesh=mesh,
        scratch_types=[
            pltpu.VMEM((NCHUNK, CH), i32),
            pltpu.VMEM((NCHUNK, CH), i32),
            [pltpu.VMEM((CH, D), f32)] * 4,
            [pltpu.VMEM((CH, D), f32)] * 4,
            pltpu.SemaphoreType.DMA,
            pltpu.SemaphoreType.DMA,
            pltpu.SemaphoreType.DMA,
            pltpu.SemaphoreType.DMA,
        ],
    )
    return fn(h, pp, src, dst)


def _scatter_body(has_pos, *refs):
    if has_pos:
        (msg_hbm, posw_hbm, dst_hbm, zn_hbm, mp_out, pp_out,
         acc, bmsg0, bmsg1, bidx0, bidx1, lsem0, lsem1) = refs
    else:
        (msg_hbm, dst_hbm, zn_hbm, mp_out,
         acc, bmsg0, bmsg1, bidx0, bidx1, lsem0, lsem1) = refs
    c = lax.axis_index("c")
    s = lax.axis_index("s")
    r0 = s * RS
    wid = s * NC + c
    wbase = wid * EW

    def one_pass(data_hbm, out_hbm):
        pltpu.sync_copy(zn_hbm.at[pl.ds(r0, RS)], acc.at[pl.ds(r0, RS)])

        @pl.when(s == 0)
        def _():
            pltpu.sync_copy(zn_hbm.at[pl.ds(NS * RS, RT)],
                            acc.at[pl.ds(NS * RS, RT)])

        plsc.subcore_barrier()

        sets = ((bmsg0, bidx0, lsem0), (bmsg1, bidx1, lsem1))

        def issue_load(k, st):
            bm, bi, ls = st
            base = pl.multiple_of(wbase + k * CH, CH)
            pltpu.async_copy(data_hbm.at[pl.ds(base, CH)], bm, ls)
            pltpu.async_copy(dst_hbm.at[pl.ds(base, CH)], bi, ls)

        def do_scatter(k, st):
            bm, bi, ls = st
            base = pl.multiple_of(wbase + k * CH, CH)
            pltpu.make_async_copy(data_hbm.at[pl.ds(base, CH)], bm, ls).wait()
            pltpu.make_async_copy(dst_hbm.at[pl.ds(base, CH)], bi, ls).wait()
            pltpu.sync_copy(bm, acc.at[bi], add=True)

        issue_load(0, sets[0])
        issue_load(1, sets[1])

        def body(g, carry):
            k0 = 2 * g
            do_scatter(k0, sets[0])
            issue_load(k0 + 2, sets[0])
            do_scatter(k0 + 1, sets[1])

            @pl.when(k0 + 3 < NCHUNK)
            def _():
                issue_load(k0 + 3, sets[1])

            return carry

        lax.fori_loop(0, (NCHUNK - 1) // 2, body, 0)
        do_scatter(NCHUNK - 1, sets[0])

        plsc.subcore_barrier()
        pltpu.sync_copy(acc.at[pl.ds(r0, RS)], out_hbm.at[c, pl.ds(r0, RS)])

        @pl.when(s == 0)
        def _():
            pltpu.sync_copy(acc.at[pl.ds(NS * RS, RT)],
                            out_hbm.at[c, pl.ds(NS * RS, RT)])

    one_pass(msg_hbm, mp_out)
    if has_pos:
        one_pass(posw_hbm, pp_out)


def _scatter(msg, dst, zn, posw=None):
    mesh = plsc.VectorSubcoreMesh(core_axis_name="c", subcore_axis_name="s")
    has_pos = posw is not None
    out_type = [jax.ShapeDtypeStruct((NC, N, D), f32)]
    args = [msg]
    if has_pos:
        out_type.append(jax.ShapeDtypeStruct((NC, N, D), f32))
        args.append(posw)
    args += [dst, zn]
    scratch = [
        pltpu.VMEM_SHARED((N, D), f32),
        pltpu.VMEM((CH, D), f32),
        pltpu.VMEM((CH, D), f32),
        pltpu.VMEM((CH,), i32),
        pltpu.VMEM((CH,), i32),
        pltpu.SemaphoreType.DMA,
        pltpu.SemaphoreType.DMA,
    ]
    fn = pl.kernel(
        functools.partial(_scatter_body, has_pos),
        out_type=out_type,
        mesh=mesh,
        scratch_types=scratch,
    )
    return fn(*args)


def _edge_body(has_pos, hd_ref, hs_ref, pd_ref, ps_ref,
               w1d_ref, w1s_ref, w1x_ref, b1_ref, g1_ref, be1_ref,
               w2_ref, b2_ref, g2_ref, be2_ref, *rest):
    dvec = pd_ref[...] - ps_ref[...]
    dist = jnp.sqrt(jnp.sum(dvec * dvec, axis=1, keepdims=True))
    t = (jnp.dot(hd_ref[...], w1d_ref[...], preferred_element_type=f32)
         + jnp.dot(hs_ref[...], w1s_ref[...], preferred_element_type=f32)
         + dist * w1x_ref[...] + b1_ref[...])
    x1 = jax.nn.relu(_ln(t, g1_ref[...], be1_ref[...]))
    msg = jax.nn.relu(_ln(jnp.dot(x1, w2_ref[...], preferred_element_type=f32)
                          + b2_ref[...], g2_ref[...], be2_ref[...]))
    if has_pos:
        (pw1_ref, pb1_ref, pg_ref, pbe_ref, pw2_ref, pb2_ref,
         msg_ref, posw_ref) = rest
        msg_ref[...] = msg
        y = jax.nn.relu(_ln(jnp.dot(msg, pw1_ref[...], preferred_element_type=f32)
                            + pb1_ref[...], pg_ref[...], pbe_ref[...]))
        sval = jnp.sum(y * pw2_ref[...], axis=1, keepdims=True) + pb2_ref[...]
        lane = lax.broadcasted_iota(i32, (BE, D), 1)
        posw_ref[...] = (jnp.where(lane < 3, dvec * sval, 0.0)
                         + (lane == 3).astype(f32))
    else:
        (msg_ref,) = rest
        msg_ref[...] = msg


def _edge(hd, hs, pd, ps, w1d, w1s, w1x, b1, g1, be1, w2, b2, g2, be2,
          pos_head=None):
    wspec = pl.BlockSpec((D, D), lambda i: (0, 0))
    rspec = pl.BlockSpec((1, D), lambda i: (0, 0))
    in_specs = [
        pl.BlockSpec((BE, D), lambda i: (i, 0)),
        pl.BlockSpec((BE, D), lambda i: (i, 0)),
        pl.BlockSpec((BE, D), lambda i: (i, 0)),
        pl.BlockSpec((BE, D), lambda i: (i, 0)),
        wspec, wspec, rspec, rspec, rspec, rspec,
        wspec, rspec, rspec, rspec,
    ]
    args = [hd, hs, pd, ps, w1d, w1s, w1x, b1, g1, be1, w2, b2, g2, be2]
    if pos_head is not None:
        pw1, pb1, pg, pbe, pw2, pb2 = pos_head
        in_specs += [wspec, rspec, rspec, rspec, rspec,
                     pl.BlockSpec((1, 1), lambda i: (0, 0))]
        args += [pw1, pb1, pg, pbe, pw2, pb2]
        out_specs = [pl.BlockSpec((BE, D), lambda i: (i, 0)),
                     pl.BlockSpec((BE, D), lambda i: (i, 0))]
        out_shape = [jax.ShapeDtypeStruct((E, D), f32),
                     jax.ShapeDtypeStruct((E, D), f32)]
    else:
        out_specs = pl.BlockSpec((BE, D), lambda i: (i, 0))
        out_shape = jax.ShapeDtypeStruct((E, D), f32)
    return pl.pallas_call(
        functools.partial(_edge_body, pos_head is not None),
        grid=(E // BE,),
        in_specs=in_specs,
        out_specs=out_specs,
        out_shape=out_shape,
    )(*args)


def _node_body(has_pos, h_ref, mp_ref,
               w1h_ref, w1m_ref, b1_ref, g1_ref, be1_ref,
               w2_ref, b2_ref, g2_ref, be2_ref, *rest):
    h = h_ref[...]
    ma = mp_ref[0] + mp_ref[1]
    t = (jnp.dot(h, w1h_ref[...], preferred_element_type=f32)
         + jnp.dot(ma, w1m_ref[...], preferred_element_type=f32) + b1_ref[...])
    u = jax.nn.relu(_ln(t, g1_ref[...], be1_ref[...]))
    upd = jax.nn.relu(_ln(jnp.dot(u, w2_ref[...], preferred_element_type=f32)
                          + b2_ref[...], g2_ref[...], be2_ref[...]))
    if has_pos:
        pp_ref, pos_ref, ho_ref, po_ref = rest
        ho_ref[...] = h + upd
        psum = pp_ref[0] + pp_ref[1]
        lane = lax.broadcasted_iota(i32, (BN, D), 1)
        cnt = jnp.sum(jnp.where(lane == 3, psum, 0.0), axis=1, keepdims=True)
        po_ref[...] = pos_ref[...] + (
            jnp.where(lane < 3, psum, 0.0) / jnp.maximum(cnt, 1.0))
    else:
        (ho_ref,) = rest
        ho_ref[...] = h + upd


def _node(h, mparts, w1h, w1m, b1, g1, be1, w2, b2, g2, be2,
          pos_part=None):
    wspec = pl.BlockSpec((D, D), lambda i: (0, 0))
    rspec = pl.BlockSpec((1, D), lambda i: (0, 0))
    in_specs = [
        pl.BlockSpec((BN, D), lambda i: (i, 0)),
        pl.BlockSpec((NC, BN, D), lambda i: (0, i, 0)),
        wspec, wspec, rspec, rspec, rspec, wspec, rspec, rspec, rspec,
    ]
    args = [h, mparts, w1h, w1m, b1, g1, be1, w2, b2, g2, be2]
    if pos_part is not None:
        pparts, pp = pos_part
        in_specs += [pl.BlockSpec((NC, BN, D), lambda i: (0, i, 0)),
                     pl.BlockSpec((BN, D), lambda i: (i, 0))]
        args += [pparts, pp]
        out_specs = [pl.BlockSpec((BN, D), lambda i: (i, 0)),
                     pl.BlockSpec((BN, D), lambda i: (i, 0))]
        out_shape = [jax.ShapeDtypeStruct((N, D), f32),
                     jax.ShapeDtypeStruct((N, D), f32)]
    else:
        out_specs = pl.BlockSpec((BN, D), lambda i: (i, 0))
        out_shape = jax.ShapeDtypeStruct((N, D), f32)
    return pl.pallas_call(
        functools.partial(_node_body, pos_part is not None),
        grid=(N // BN,),
        in_specs=in_specs,
        out_specs=out_specs,
        out_shape=out_shape,
    )(*args)


def _pool_body(h_ref, b_ref, w1_ref, b1_ref, w2t_ref, b2_ref, out_ref):
    b = b_ref[...]
    oh = (lax.broadcasted_iota(i32, (G, N), 0) == b).astype(f32)
    pooled = jnp.dot(oh, h_ref[...], preferred_element_type=f32)
    r = jax.nn.relu(jnp.dot(pooled, w1_ref[...], preferred_element_type=f32)
                    + b1_ref[...])
    out_ref[...] = jnp.sum(r * w2t_ref[...], axis=1, keepdims=True) + b2_ref[...]


def _pool(h, batch2, w1, b1, w2t, b2):
    return pl.pallas_call(
        _pool_body,
        grid=(1,),
        in_specs=[
            pl.BlockSpec((N, D), lambda i: (0, 0)),
            pl.BlockSpec((1, N), lambda i: (0, 0)),
            pl.BlockSpec((D, D), lambda i: (0, 0)),
            pl.BlockSpec((1, D), lambda i: (0, 0)),
            pl.BlockSpec((1, D), lambda i: (0, 0)),
            pl.BlockSpec((1, 1), lambda i: (0, 0)),
        ],
        out_specs=pl.BlockSpec((G, 1), lambda i: (0, 0)),
        out_shape=jax.ShapeDtypeStruct((G, 1), f32),
    )(h, batch2, w1, b1, w2t, b2)


def kernel(atoms, pos, edge_index, batch, emb,
           msg_W1, msg_b1, msg_g1, msg_be1, msg_W2, msg_b2, msg_g2, msg_be2,
           pos_W1, pos_b1, pos_g, pos_be, pos_W2, pos_b2,
           upd_W1, upd_b1, upd_g1, upd_be1, upd_W2, upd_b2, upd_g2, upd_be2,
           pred_W1, pred_b1, pred_W2, pred_b2):
    src = edge_index[0].astype(i32)
    dst = edge_index[1].astype(i32)
    posp = jnp.pad(pos.astype(f32), ((0, 0), (0, D - 3)))
    embp = jnp.pad(emb, ((0, D - emb.shape[0]), (0, 0)))
    atoms3 = atoms.astype(i32).reshape(N // BN, BN, 1)
    batch2 = batch.astype(i32).reshape(1, N)
    zn = jnp.zeros((N, D), f32)

    def r1(a):
        return a.reshape(1, D)

    h = _embed(atoms3, embp)
    pp = posp
    for l in range(L):
        w1d = msg_W1[l, :D]
        w1s = msg_W1[l, D:2 * D]
        w1x = msg_W1[l, 2 * D:2 * D + 1]
        hd, hs, pd, ps = _gather(h, pp, src, dst)
        if l < L - 1:
            pos_head = (pos_W1[l], r1(pos_b1[l]), r1(pos_g[l]), r1(pos_be[l]),
                        pos_W2[l].reshape(1, D), pos_b2[l].reshape(1, 1))
            msg, posw = _edge(hd, hs, pd, ps, w1d, w1s, w1x,
                              r1(msg_b1[l]), r1(msg_g1[l]), r1(msg_be1[l]),
                              msg_W2[l], r1(msg_b2[l]), r1(msg_g2[l]),
                              r1(msg_be2[l]), pos_head=pos_head)
            mparts, pparts = _scatter(msg, dst, zn, posw=posw)
            h, pp = _node(h, mparts,
                          upd_W1[l, :D], upd_W1[l, D:], r1(upd_b1[l]),
                          r1(upd_g1[l]), r1(upd_be1[l]), upd_W2[l],
                          r1(upd_b2[l]), r1(upd_g2[l]), r1(upd_be2[l]),
                          pos_part=(pparts, pp))
        else:
            msg = _edge(hd, hs, pd, ps, w1d, w1s, w1x,
                        r1(msg_b1[l]), r1(msg_g1[l]), r1(msg_be1[l]),
                        msg_W2[l], r1(msg_b2[l]), r1(msg_g2[l]),
                        r1(msg_be2[l]))
            (mparts,) = _scatter(msg, dst, zn)
            h = _node(h, mparts,
                      upd_W1[l, :D], upd_W1[l, D:], r1(upd_b1[l]),
                      r1(upd_g1[l]), r1(upd_be1[l]), upd_W2[l],
                      r1(upd_b2[l]), r1(upd_g2[l]), r1(upd_be2[l]))
    return _pool(h, batch2, pred_W1, pred_b1.reshape(1, D),
                 pred_W2.reshape(1, D), pred_b2.reshape(1, 1))
